# baseline (speedup 1.0000x reference)
import jax
import jax.numpy as jnp
from jax.experimental import pallas as pl
from jax.experimental.pallas import tpu as pltpu

_NQ = 4
_K = 1024
_D = 64


def _rvq_block_kernel(x_ref, cb_ref, cn_ref, quant_ref, nn_ref, counts_ref,
                      loss_ref):
    j = pl.program_id(0)

    @pl.when(j == 0)
    def _init():
        counts_ref[...] = jnp.zeros_like(counts_ref)
        loss_ref[...] = jnp.zeros_like(loss_ref)

    x = x_ref[...]
    b = x.shape[0]
    r = x
    qsum = jnp.zeros_like(x)
    col_iota = jax.lax.broadcasted_iota(jnp.int32, (b, _K), 1)
    loss_sum = jnp.float32(0.0)
    nn_cols = []
    cnt_rows = []
    for i in range(_NQ):
        cb = cb_ref[i]
        cn = cn_ref[i:i + 1, :]
        s = jax.lax.dot_general(r, cb, (((1,), (1,)), ((), ())),
                                preferred_element_type=jnp.float32)
        rn = jnp.sum(r * r, axis=1, keepdims=True)
        d2 = rn - 2.0 * s + cn
        m = jnp.min(d2, axis=1, keepdims=True)
        idx = jnp.min(jnp.where(d2 == m, col_iota, _K), axis=1,
                      keepdims=True)
        oh = col_iota == idx
        onehot = oh.astype(jnp.float32)
        q = jax.lax.dot_general(onehot, cb, (((1,), (0,)), ((), ())),
                                precision=jax.lax.Precision.HIGHEST,
                                preferred_element_type=jnp.float32)
        q_st = r + (q - r)
        qsum = qsum + q_st
        diff = r - q
        e = diff * diff
        loss_sum = loss_sum + jnp.sum(jnp.mean(e + 0.25 * e, axis=1))
        nn_cols.append(idx)
        cnt_rows.append(jnp.sum(oh.astype(jnp.int32), axis=0, keepdims=True))
        r = r - q_st
    quant_ref[...] = qsum
    nn_ref[...] = jnp.concatenate(nn_cols, axis=1)
    counts_ref[...] += jnp.concatenate(cnt_rows, axis=0)
    loss_ref[...] += loss_sum.reshape(1, 1)


def kernel(inputs, codebooks):
    shape = inputs.shape
    d = shape[-1]
    flat = inputs.reshape(-1, d)
    n = flat.shape[0]
    nq, k, _ = codebooks.shape
    cnorm = jnp.stack(
        [jnp.sum(codebooks[i] * codebooks[i], axis=1) for i in range(nq)],
        axis=0)
    blk = 512
    grid = (n // blk,)
    quant, nn, counts, loss = pl.pallas_call(
        _rvq_block_kernel,
        grid=grid,
        in_specs=[
            pl.BlockSpec((blk, d), lambda j: (j, 0)),
            pl.BlockSpec((nq, k, d), lambda j: (0, 0, 0)),
            pl.BlockSpec((nq, k), lambda j: (0, 0)),
        ],
        out_specs=[
            pl.BlockSpec((blk, d), lambda j: (j, 0)),
            pl.BlockSpec((blk, nq), lambda j: (j, 0)),
            pl.BlockSpec((nq, k), lambda j: (0, 0)),
            pl.BlockSpec((1, 1), lambda j: (0, 0)),
        ],
        out_shape=[
            jax.ShapeDtypeStruct((n, d), jnp.float32),
            jax.ShapeDtypeStruct((n, nq), jnp.int32),
            jax.ShapeDtypeStruct((nq, k), jnp.int32),
            jax.ShapeDtypeStruct((1, 1), jnp.float32),
        ],
        compiler_params=pltpu.CompilerParams(
            dimension_semantics=("arbitrary",)),
    )(flat, codebooks, cnorm)
    quantized = quant.reshape(shape)
    qloss = loss[0, 0] / jnp.float32(n)
    qloss_out = jnp.full(shape[:-1] + (1,), qloss, dtype=jnp.float32)
    nn_idx = nn.T.reshape((nq,) + shape[:-1])
    codebooks_out = codebooks.reshape(-1, d)
    return quantized, qloss_out, nn_idx, codebooks_out, counts


# speedup vs baseline: 1.5458x; 1.5458x over previous
import jax
import jax.numpy as jnp
from jax.experimental import pallas as pl
from jax.experimental.pallas import tpu as pltpu

_NQ = 4
_K = 1024
_D = 64


def _rvq_block_kernel(x_ref, cb_ref, cb3_ref, cn_ref, quant_ref, nn_ref,
                      counts_ref, loss_ref):
    j = pl.program_id(0)

    @pl.when(j == 0)
    def _init():
        counts_ref[...] = jnp.zeros_like(counts_ref)
        loss_ref[...] = jnp.zeros_like(loss_ref)

    x = x_ref[...]
    b = x.shape[0]
    r = x
    qsum = jnp.zeros_like(x)
    col_iota = jax.lax.broadcasted_iota(jnp.int32, (b, _K), 1)
    loss_sum = jnp.float32(0.0)
    nn_cols = []
    cnt_rows = []
    for i in range(_NQ):
        cb = cb_ref[i]
        cn = cn_ref[i:i + 1, :]
        s = jax.lax.dot_general(r, cb, (((1,), (1,)), ((), ())),
                                preferred_element_type=jnp.float32)
        rn = jnp.sum(r * r, axis=1, keepdims=True)
        d2 = rn - 2.0 * s + cn
        m = jnp.min(d2, axis=1, keepdims=True)
        idx = jnp.min(jnp.where(d2 == m, col_iota, _K), axis=1,
                      keepdims=True)
        oh = col_iota == idx
        onehot = oh.astype(jnp.float32)
        q0 = jax.lax.dot_general(onehot, cb3_ref[3 * i],
                                 (((1,), (0,)), ((), ())),
                                 preferred_element_type=jnp.float32)
        q1 = jax.lax.dot_general(onehot, cb3_ref[3 * i + 1],
                                 (((1,), (0,)), ((), ())),
                                 preferred_element_type=jnp.float32)
        q2 = jax.lax.dot_general(onehot, cb3_ref[3 * i + 2],
                                 (((1,), (0,)), ((), ())),
                                 preferred_element_type=jnp.float32)
        q = (q0 + q1) + q2
        q_st = r + (q - r)
        qsum = qsum + q_st
        diff = r - q
        e = diff * diff
        loss_sum = loss_sum + jnp.sum(jnp.mean(e + 0.25 * e, axis=1))
        nn_cols.append(idx)
        cnt_rows.append(jnp.sum(oh.astype(jnp.int32), axis=0, keepdims=True))
        r = r - q_st
    quant_ref[...] = qsum
    nn_ref[...] = jnp.concatenate(nn_cols, axis=1)
    counts_ref[...] += jnp.concatenate(cnt_rows, axis=0)
    loss_ref[...] += loss_sum.reshape(1, 1)


def kernel(inputs, codebooks):
    shape = inputs.shape
    d = shape[-1]
    flat = inputs.reshape(-1, d)
    n = flat.shape[0]
    nq, k, _ = codebooks.shape
    cnorm = jnp.stack(
        [jnp.sum(codebooks[i] * codebooks[i], axis=1) for i in range(nq)],
        axis=0)
    mask = jnp.uint32(0xFFFF0000)
    u = codebooks
    hi = jax.lax.bitcast_convert_type(
        jax.lax.bitcast_convert_type(u, jnp.uint32) & mask, jnp.float32)
    r1 = u - hi
    mid = jax.lax.bitcast_convert_type(
        jax.lax.bitcast_convert_type(r1, jnp.uint32) & mask, jnp.float32)
    lo = r1 - mid
    cb3 = jnp.stack([hi, mid, lo], axis=1).reshape(3 * nq, k, d)
    blk = 512
    grid = (n // blk,)
    quant, nn, counts, loss = pl.pallas_call(
        _rvq_block_kernel,
        grid=grid,
        in_specs=[
            pl.BlockSpec((blk, d), lambda j: (j, 0)),
            pl.BlockSpec((nq, k, d), lambda j: (0, 0, 0)),
            pl.BlockSpec((3 * nq, k, d), lambda j: (0, 0, 0)),
            pl.BlockSpec((nq, k), lambda j: (0, 0)),
        ],
        out_specs=[
            pl.BlockSpec((blk, d), lambda j: (j, 0)),
            pl.BlockSpec((blk, nq), lambda j: (j, 0)),
            pl.BlockSpec((nq, k), lambda j: (0, 0)),
            pl.BlockSpec((1, 1), lambda j: (0, 0)),
        ],
        out_shape=[
            jax.ShapeDtypeStruct((n, d), jnp.float32),
            jax.ShapeDtypeStruct((n, nq), jnp.int32),
            jax.ShapeDtypeStruct((nq, k), jnp.int32),
            jax.ShapeDtypeStruct((1, 1), jnp.float32),
        ],
        compiler_params=pltpu.CompilerParams(
            dimension_semantics=("arbitrary",)),
    )(flat, codebooks, cb3, cnorm)
    quantized = quant.reshape(shape)
    qloss = loss[0, 0] / jnp.float32(n)
    qloss_out = jnp.full(shape[:-1] + (1,), qloss, dtype=jnp.float32)
    nn_idx = nn.T.reshape((nq,) + shape[:-1])
    codebooks_out = codebooks.reshape(-1, d)
    return quantized, qloss_out, nn_idx, codebooks_out, counts


# speedup vs baseline: 2.2357x; 1.4463x over previous
import jax
import jax.numpy as jnp
from jax.experimental import pallas as pl
from jax.experimental.pallas import tpu as pltpu

_NQ = 4
_K = 1024
_D = 64


def _rvq_block_kernel(x_ref, cb_ref, cb3_ref, cn_ref, quant_ref, nn_ref,
                      counts_ref, loss_ref):
    j = pl.program_id(0)

    @pl.when(j == 0)
    def _init():
        counts_ref[...] = jnp.zeros_like(counts_ref)
        loss_ref[...] = jnp.zeros_like(loss_ref)

    x = x_ref[...]
    b = x.shape[0]
    r = x
    qsum = jnp.zeros_like(x)
    col_iota = jax.lax.broadcasted_iota(jnp.int32, (b, _K), 1)
    loss_sum = jnp.float32(0.0)
    nn_cols = []
    cnt_rows = []
    for i in range(_NQ):
        cb = cb_ref[i]
        cn = cn_ref[i:i + 1, :]
        s = jax.lax.dot_general(r, cb, (((1,), (1,)), ((), ())),
                                preferred_element_type=jnp.float32)
        rn = jnp.sum(r * r, axis=1, keepdims=True)
        d2 = rn - 2.0 * s + cn
        m = jnp.min(d2, axis=1, keepdims=True)
        idx = jnp.min(jnp.where(d2 == m, col_iota, _K), axis=1,
                      keepdims=True)
        oh = col_iota == idx
        onehot = oh.astype(jnp.float32)
        q3 = jax.lax.dot_general(onehot, cb3_ref[i],
                                 (((1,), (0,)), ((), ())),
                                 preferred_element_type=jnp.float32)
        q = (q3[:, :_D] + q3[:, _D:2 * _D]) + q3[:, 2 * _D:]
        q_st = r + (q - r)
        qsum = qsum + q_st
        diff = r - q
        e = diff * diff
        loss_sum = loss_sum + jnp.sum(jnp.mean(e + 0.25 * e, axis=1))
        nn_cols.append(idx)
        cnt_rows.append(jnp.sum(oh.astype(jnp.int32), axis=0, keepdims=True))
        r = r - q_st
    quant_ref[...] = qsum
    nn_ref[...] = jnp.concatenate(nn_cols, axis=1)
    counts_ref[...] += jnp.concatenate(cnt_rows, axis=0)
    loss_ref[...] += loss_sum.reshape(1, 1)


def kernel(inputs, codebooks):
    shape = inputs.shape
    d = shape[-1]
    flat = inputs.reshape(-1, d)
    n = flat.shape[0]
    nq, k, _ = codebooks.shape
    cnorm = jnp.stack(
        [jnp.sum(codebooks[i] * codebooks[i], axis=1) for i in range(nq)],
        axis=0)
    mask = jnp.uint32(0xFFFF0000)
    u = codebooks
    hi = jax.lax.bitcast_convert_type(
        jax.lax.bitcast_convert_type(u, jnp.uint32) & mask, jnp.float32)
    r1 = u - hi
    mid = jax.lax.bitcast_convert_type(
        jax.lax.bitcast_convert_type(r1, jnp.uint32) & mask, jnp.float32)
    lo = r1 - mid
    cb3 = jnp.concatenate([hi, mid, lo], axis=-1)
    blk = 512
    grid = (n // blk,)
    quant, nn, counts, loss = pl.pallas_call(
        _rvq_block_kernel,
        grid=grid,
        in_specs=[
            pl.BlockSpec((blk, d), lambda j: (j, 0)),
            pl.BlockSpec((nq, k, d), lambda j: (0, 0, 0)),
            pl.BlockSpec((nq, k, 3 * d), lambda j: (0, 0, 0)),
            pl.BlockSpec((nq, k), lambda j: (0, 0)),
        ],
        out_specs=[
            pl.BlockSpec((blk, d), lambda j: (j, 0)),
            pl.BlockSpec((blk, nq), lambda j: (j, 0)),
            pl.BlockSpec((nq, k), lambda j: (0, 0)),
            pl.BlockSpec((1, 1), lambda j: (0, 0)),
        ],
        out_shape=[
            jax.ShapeDtypeStruct((n, d), jnp.float32),
            jax.ShapeDtypeStruct((n, nq), jnp.int32),
            jax.ShapeDtypeStruct((nq, k), jnp.int32),
            jax.ShapeDtypeStruct((1, 1), jnp.float32),
        ],
        compiler_params=pltpu.CompilerParams(
            dimension_semantics=("arbitrary",)),
    )(flat, codebooks, cb3, cnorm)
    quantized = quant.reshape(shape)
    qloss = loss[0, 0] / jnp.float32(n)
    qloss_out = jnp.full(shape[:-1] + (1,), qloss, dtype=jnp.float32)
    nn_idx = nn.T.reshape((nq,) + shape[:-1])
    codebooks_out = codebooks.reshape(-1, d)
    return quantized, qloss_out, nn_idx, codebooks_out, counts
